# Initial kernel scaffold; baseline (speedup 1.0000x reference)
import jax
import jax.numpy as jnp
from jax.experimental import pallas as pl


def kernel(pos_u, pos_v, neg_v, u_weight, v_weight):
    raise NotImplementedError("write your pallas kernel here")


# speedup vs baseline: 1.7156x; 1.7156x over previous
import functools

import jax
import jax.numpy as jnp
from jax import lax
from jax.experimental import pallas as pl
from jax.experimental.pallas import tpu as pltpu
from jax.experimental.pallas import tpu_sc as plsc

NC = 2
NS = 16
NW = NC * NS
N_NEG = 5
CHUNK = 128


def _sc_scores(b_per_w, n_chunk, d):
    mesh = plsc.VectorSubcoreMesh(core_axis_name="c", subcore_axis_name="s")
    batch = b_per_w * NW

    @functools.partial(
        pl.kernel,
        mesh=mesh,
        out_type=jax.ShapeDtypeStruct((8, batch), jnp.float32),
        compiler_params=pltpu.CompilerParams(needs_layout_passes=False,
                                             use_tc_tiling_on_sc=False),
        scratch_types=[
            pltpu.VMEM((CHUNK,), jnp.int32),
            pltpu.VMEM((CHUNK,), jnp.int32),
            pltpu.VMEM((N_NEG * CHUNK,), jnp.int32),
            pltpu.VMEM((CHUNK, d), jnp.float32),
            pltpu.VMEM((CHUNK, d), jnp.float32),
            pltpu.VMEM((CHUNK * N_NEG, d), jnp.float32),
            pltpu.VMEM((6, b_per_w), jnp.float32),
            pltpu.SemaphoreType.DMA,
        ],
    )
    def k(pos_u1, pos_v1, neg1, u_w, v_w, out, idx_u, idx_v, idx_n,
          u_buf, v_buf, n_buf, acc_buf, sem):
        wid = lax.axis_index("s") * NC + lax.axis_index("c")
        base = wid * b_per_w
        iota16 = lax.iota(jnp.int32, 16)

        for c in range(n_chunk):
            off0 = base + c * CHUNK
            pltpu.sync_copy(pos_u1.at[pl.ds(off0, CHUNK)], idx_u)
            pltpu.sync_copy(pos_v1.at[pl.ds(off0, CHUNK)], idx_v)
            pltpu.sync_copy(neg1.at[pl.ds(off0 * N_NEG, CHUNK * N_NEG)], idx_n)
            cps = [
                pltpu.async_copy(u_w.at[idx_u], u_buf, sem),
                pltpu.async_copy(v_w.at[idx_v], v_buf, sem),
            ]
            for j in range(N_NEG):
                cps.append(pltpu.async_copy(
                    v_w.at[idx_n.at[pl.ds(j * CHUNK, CHUNK)]],
                    n_buf.at[pl.ds(j * CHUNK, CHUNK)], sem))
            for cp in cps:
                cp.wait()

            nseg = d // 16
            masks = [iota16 == i for i in range(16)]

            def body(g, _):
                accs = [jnp.zeros((16,), jnp.float32) for _ in range(6)]
                for i in range(16):
                    bl = g * 16 + i
                    us = [u_buf[bl, pl.ds(s * 16, 16)] for s in range(nseg)]
                    vs = [v_buf[bl, pl.ds(s * 16, 16)] for s in range(nseg)]
                    acc = us[0] * vs[0]
                    for s in range(1, nseg):
                        acc = acc + us[s] * vs[s]
                    accs[0] = jnp.where(masks[i], jnp.sum(acc), accs[0])
                    for n in range(N_NEG):
                        r = bl * N_NEG + n
                        ms = [n_buf[r, pl.ds(s * 16, 16)] for s in range(nseg)]
                        nacc = ms[0] * us[0]
                        for s in range(1, nseg):
                            nacc = nacc + ms[s] * us[s]
                        accs[1 + n] = jnp.where(masks[i], jnp.sum(nacc),
                                                accs[1 + n])
                off = c * CHUNK + g * 16
                for r in range(6):
                    acc_buf[r, pl.ds(off, 16)] = accs[r]
                return 0

            lax.fori_loop(0, CHUNK // 16, body, 0)

        for r in range(6):
            pltpu.sync_copy(acc_buf.at[r], out.at[r, pl.ds(base, b_per_w)])

    return k


def _tc_reduce(scores):
    def body(s_ref, o_ref):
        s = s_ref[...]
        rid = lax.broadcasted_iota(jnp.int32, s.shape, 0)
        valid = rid < 6
        sign = jnp.where(rid == 0, 1.0, -1.0)
        x = jnp.where(valid, s * sign, 0.0)
        vals = jax.nn.log_sigmoid(x)
        o_ref[0, 0] = -jnp.sum(jnp.where(valid, vals, 0.0))

    return pl.pallas_call(
        body,
        out_shape=jax.ShapeDtypeStruct((1, 1), jnp.float32),
        in_specs=[pl.BlockSpec(memory_space=pltpu.VMEM)],
        out_specs=pl.BlockSpec(memory_space=pltpu.SMEM),
    )(scores)


def kernel(pos_u, pos_v, neg_v, u_weight, v_weight):
    batch = pos_u.shape[0]
    d = u_weight.shape[1]
    b_per_w = batch // NW
    n_chunk = b_per_w // CHUNK

    pos_u1 = pos_u.astype(jnp.int32)
    pos_v1 = pos_v.astype(jnp.int32)
    neg1 = neg_v.astype(jnp.int32).reshape(batch * N_NEG)

    scores = _sc_scores(b_per_w, n_chunk, d)(pos_u1, pos_v1, neg1,
                                             u_weight, v_weight)
    return _tc_reduce(scores)[0, 0]
